# baseline (speedup 1.0000x reference)
import functools

import jax
import jax.numpy as jnp
from jax import lax
from jax.experimental import pallas as pl
from jax.experimental.pallas import tpu as pltpu
from jax.experimental.pallas import tpu_sc as plsc

N = 10000
E = 320000
D = 128
HID = 128
A = 16384

NC = 2
NS = 16
NW = NC * NS
NP = 10240
ROWS_PER_TILE = NP // NS
WIN = 128
WPW = 80
CH = 16
NCH = WPW // CH
NBUF = 2
E_PAD = NW * WPW * WIN
PAD = E_PAD - E

_mesh = plsc.VectorSubcoreMesh(core_axis_name="c", subcore_axis_name="s")


def _worker_id():
    return lax.axis_index("s") * NC + lax.axis_index("c")


@functools.partial(
    pl.kernel,
    out_type=jax.ShapeDtypeStruct((NC, 1, NP), jnp.float32),
    mesh=_mesh,
    scratch_types=[
        pltpu.VMEM((WPW, WIN), jnp.int32),
        pltpu.VMEM((WIN,), jnp.float32),
        pltpu.VMEM((640,), jnp.float32),
        pltpu.VMEM_SHARED((NP,), jnp.float32),
        pltpu.SemaphoreType.DMA,
    ],
)
def _sc_deg(dst3, out, idx_d, ones, zbuf, acc, sa):
    c = lax.axis_index("c")
    s = lax.axis_index("s")
    wid = _worker_id()

    def _z(i, _):
        zbuf[pl.ds(i * 16, 16)] = jnp.zeros((16,), jnp.float32)
        return 0
    lax.fori_loop(0, 640 // 16, _z, 0)

    def _o(i, _):
        ones[pl.ds(i * 16, 16)] = jnp.ones((16,), jnp.float32)
        return 0
    lax.fori_loop(0, WIN // 16, _o, 0)

    pltpu.sync_copy(zbuf.at[pl.ds(0, ROWS_PER_TILE)],
                    acc.at[pl.ds(s * ROWS_PER_TILE, ROWS_PER_TILE)])
    plsc.subcore_barrier()

    pltpu.sync_copy(dst3.at[wid], idx_d)

    def _step(k, _):
        pltpu.async_copy(ones, acc.at[idx_d.at[k]], sa, add=True)
        return 0
    lax.fori_loop(0, WPW, _step, 0)

    def _drain(k, _):
        pltpu.make_async_copy(ones, acc.at[idx_d.at[k]], sa).wait()
        return 0
    lax.fori_loop(0, WPW, _drain, 0)

    plsc.subcore_barrier()
    @pl.when(s == 0)
    def _():
        pltpu.sync_copy(acc, out.at[c, 0])


@functools.partial(
    pl.kernel,
    out_type=jax.ShapeDtypeStruct((NC, NP, D), jnp.float32),
    mesh=_mesh,
    scratch_types=[
        pltpu.VMEM((2, CH, WIN), jnp.int32),
        pltpu.VMEM((2, CH, WIN), jnp.int32),
        pltpu.VMEM((NBUF, WIN, D), jnp.float32),
        pltpu.VMEM_SHARED((NP, D), jnp.float32),
        pltpu.SemaphoreType.DMA,
        pltpu.SemaphoreType.DMA,
        pltpu.SemaphoreType.DMA,
    ],
)
def _sc_agg(xws, src4, dst4, out, idxs, idxd, rows, acc, sg0, sg1, si):
    c = lax.axis_index("c")
    s = lax.axis_index("s")
    wid = _worker_id()
    sems = (sg0, sg1)

    pend = (pltpu.async_copy(src4.at[wid, 0], idxs.at[0], si),
            pltpu.async_copy(dst4.at[wid, 0], idxd.at[0], si))

    def _z(i, _):
        def _z2(j, _):
            rows[0, i, pl.ds(j * 16, 16)] = jnp.zeros((16,), jnp.float32)
            return 0
        return lax.fori_loop(0, D // 16, _z2, 0)
    lax.fori_loop(0, WIN, _z, 0)

    base = s * ROWS_PER_TILE

    def _zc(i, _):
        pltpu.sync_copy(rows.at[0], acc.at[pl.ds(base + i * WIN, WIN)])
        return 0
    lax.fori_loop(0, ROWS_PER_TILE // WIN, _zc, 0)
    pend[0].wait()
    pend[1].wait()
    plsc.subcore_barrier()

    for b in range(NBUF):
        pltpu.async_copy(xws.at[idxs.at[0, b]], rows.at[b], sems[b])
    pend = (pltpu.async_copy(src4.at[wid, 1], idxs.at[1], si),
            pltpu.async_copy(dst4.at[wid, 1], idxd.at[1], si))

    for ch in range(NCH):
        p = ch % 2

        def _quad(j, _):
            for b in range(NBUF):
                w = NBUF * j + b
                pltpu.make_async_copy(xws.at[idxs.at[p, w]], rows.at[b],
                                      sems[b]).wait()
                pltpu.sync_copy(rows.at[b], acc.at[idxd.at[p, w]],
                                add=True)
                pltpu.async_copy(xws.at[idxs.at[p, w + NBUF]], rows.at[b],
                                 sems[b])
            return 0
        lax.fori_loop(0, CH // NBUF - 1, _quad, 0)

        for b in range(NBUF):
            w = CH - NBUF + b
            pltpu.make_async_copy(xws.at[idxs.at[p, w]], rows.at[b],
                                  sems[b]).wait()
            pltpu.sync_copy(rows.at[b], acc.at[idxd.at[p, w]], add=True)
            if ch + 1 < NCH:
                if b == 0:
                    pend[0].wait()
                    pend[1].wait()
                pltpu.async_copy(xws.at[idxs.at[1 - p, b]], rows.at[b],
                                 sems[b])
                if b == NBUF - 1 and ch + 2 < NCH:
                    pend = (pltpu.async_copy(src4.at[wid, ch + 2],
                                             idxs.at[p], si),
                            pltpu.async_copy(dst4.at[wid, ch + 2],
                                             idxd.at[p], si))

    plsc.subcore_barrier()
    pltpu.sync_copy(acc.at[pl.ds(base, ROWS_PER_TILE)],
                    out.at[c, pl.ds(base, ROWS_PER_TILE)])


AWIN = 128
_AROWS = A // AWIN
_APW = _AROWS // NW


@functools.partial(
    pl.kernel,
    out_type=(jax.ShapeDtypeStruct((A, D), jnp.float32),
              jax.ShapeDtypeStruct((A, D), jnp.float32)),
    mesh=_mesh,
    scratch_types=[
        pltpu.VMEM((_APW, AWIN), jnp.int32),
        pltpu.VMEM((_APW, AWIN), jnp.int32),
        pltpu.VMEM((2, AWIN, D), jnp.float32),
        pltpu.SemaphoreType.DMA,
        pltpu.SemaphoreType.DMA,
        pltpu.SemaphoreType.DMA,
        pltpu.SemaphoreType.DMA,
    ],
)
def _sc_take(h2, att3, dfd3, out_a, out_d, idx_a, idx_d, rows,
             sg0, sg1, sw0, sw1):
    wid = _worker_id()
    pltpu.sync_copy(att3.at[wid], idx_a)
    pltpu.sync_copy(dfd3.at[wid], idx_d)
    sg = (sg0, sg1)
    sw = (sw0, sw1)

    def _idx(t):
        return (idx_a if t % 2 == 0 else idx_d).at[t // 2]

    def _out(t):
        o = out_a if t % 2 == 0 else out_d
        return o.at[pl.ds((wid * _APW + t // 2) * AWIN, AWIN)]

    njobs = 2 * _APW
    pltpu.async_copy(h2.at[_idx(0)], rows.at[0], sg[0])
    for t in range(njobs):
        b = t % 2
        pltpu.make_async_copy(h2.at[_idx(t)], rows.at[b], sg[b]).wait()
        if t + 1 < njobs:
            if t >= 1:
                pltpu.make_async_copy(rows.at[1 - b], _out(t - 1),
                                      sw[1 - b]).wait()
            pltpu.async_copy(h2.at[_idx(t + 1)], rows.at[1 - b], sg[1 - b])
        pltpu.async_copy(rows.at[b], _out(t), sw[b])
    pltpu.make_async_copy(rows.at[0], _out(njobs - 2), sw[0]).wait()
    pltpu.make_async_copy(rows.at[1], _out(njobs - 1), sw[1]).wait()


_RB = 2000


def _dinv_block(dr):
    return lax.rsqrt(dr[:, 0:1] + dr[:, 1:2] + 1.0)


def _tc_xws1_body(xr, wr, dr, outr):
    dinv = _dinv_block(dr)
    outr[...] = jnp.dot(xr[...], wr[...],
                        preferred_element_type=jnp.float32) * dinv


def _tc_xws1(x, W1, degT):
    return pl.pallas_call(
        _tc_xws1_body,
        grid=(N // _RB,),
        in_specs=[
            pl.BlockSpec((_RB, D), lambda i: (i, 0)),
            pl.BlockSpec((D, HID), lambda i: (0, 0)),
            pl.BlockSpec((_RB, NC), lambda i: (i, 0)),
        ],
        out_specs=pl.BlockSpec((_RB, HID), lambda i: (i, 0)),
        out_shape=jax.ShapeDtypeStruct((N, HID), jnp.float32),
    )(x, W1, degT)


def _tc_layer_body(ar, xr, dr, wr, br, outr):
    dinv = _dinv_block(dr)
    h = jnp.maximum(dinv * (ar[0] + ar[1] + xr[...]) + br[...], 0.0)
    outr[...] = jnp.dot(h, wr[...], preferred_element_type=jnp.float32) * dinv


def _tc_layer(aggp, xws, degT, W, b):
    return pl.pallas_call(
        _tc_layer_body,
        grid=(N // _RB,),
        in_specs=[
            pl.BlockSpec((NC, _RB, HID), lambda i: (0, i, 0)),
            pl.BlockSpec((_RB, HID), lambda i: (i, 0)),
            pl.BlockSpec((_RB, NC), lambda i: (i, 0)),
            pl.BlockSpec((HID, HID), lambda i: (0, 0)),
            pl.BlockSpec((1, HID), lambda i: (0, 0)),
        ],
        out_specs=pl.BlockSpec((_RB, HID), lambda i: (i, 0)),
        out_shape=jax.ShapeDtypeStruct((N, HID), jnp.float32),
    )(aggp, xws, degT, W, b)


def _tc_h2_body(ar, xr, dr, br, outr):
    dinv = _dinv_block(dr)
    outr[...] = jnp.maximum(dinv * (ar[0] + ar[1] + xr[...]) + br[...], 0.0)


def _tc_h2(aggp, xws, degT, b):
    return pl.pallas_call(
        _tc_h2_body,
        grid=(N // _RB,),
        in_specs=[
            pl.BlockSpec((NC, _RB, HID), lambda i: (0, i, 0)),
            pl.BlockSpec((_RB, HID), lambda i: (i, 0)),
            pl.BlockSpec((_RB, NC), lambda i: (i, 0)),
            pl.BlockSpec((1, HID), lambda i: (0, 0)),
        ],
        out_specs=pl.BlockSpec((_RB, HID), lambda i: (i, 0)),
        out_shape=jax.ShapeDtypeStruct((N, HID), jnp.float32),
    )(aggp, xws, degT, b)


_AB = 2048


def _bf16_round(v):
    return v.astype(jnp.bfloat16).astype(jnp.float32)


def _tc_mlp_body(atr, dfr, nsr, wa, wb, wc, b1r, w2r, b2r, outr):
    hid = (jnp.dot(atr[...], wa[...], preferred_element_type=jnp.float32)
           + jnp.dot(dfr[...], wb[...], preferred_element_type=jnp.float32)
           + _bf16_round(nsr[...]) * _bf16_round(wc[...]) + b1r[...])
    hid = jnp.maximum(hid, 0.0)
    prod = _bf16_round(hid) * _bf16_round(w2r[...])
    outr[...] = jnp.sum(prod, axis=1, keepdims=True) + b2r[...]


def _tc_mlp(att_e, dfd_e, ns, mw1a, mw1b, mw1c, mb1, mw2r, mb2):
    return pl.pallas_call(
        _tc_mlp_body,
        grid=(A // _AB,),
        in_specs=[
            pl.BlockSpec((_AB, HID), lambda i: (i, 0)),
            pl.BlockSpec((_AB, HID), lambda i: (i, 0)),
            pl.BlockSpec((_AB, 1), lambda i: (i, 0)),
            pl.BlockSpec((HID, HID), lambda i: (0, 0)),
            pl.BlockSpec((HID, HID), lambda i: (0, 0)),
            pl.BlockSpec((1, HID), lambda i: (0, 0)),
            pl.BlockSpec((1, HID), lambda i: (0, 0)),
            pl.BlockSpec((1, HID), lambda i: (0, 0)),
            pl.BlockSpec((1, 1), lambda i: (0, 0)),
        ],
        out_specs=pl.BlockSpec((_AB, 1), lambda i: (i, 0)),
        out_shape=jax.ShapeDtypeStruct((A, 1), jnp.float32),
    )(att_e, dfd_e, ns, mw1a, mw1b, mw1c, mb1, mw2r, mb2)


def kernel(x, edge_index, action_lookup_table, W1, b1, W2, b2,
           skip_attack, skip_defend, mw1, mb1, mw2, mb2):
    pad_src = jnp.arange(PAD, dtype=jnp.int32) % N
    pad_dst = N + jnp.arange(PAD, dtype=jnp.int32) % (NP - N)
    src3 = jnp.concatenate([edge_index[0], pad_src]).reshape(NW, WPW, WIN)
    dst3 = jnp.concatenate([edge_index[1], pad_dst]).reshape(NW, WPW, WIN)
    src4 = src3.reshape(NW, NCH, CH, WIN)
    dst4 = dst3.reshape(NW, NCH, CH, WIN)

    degp = _sc_deg(dst3)
    degT = degp.reshape(NC, NP).T

    xws1 = _tc_xws1(x, W1, degT)
    aggp1 = _sc_agg(xws1, src4, dst4)
    xws2 = _tc_layer(aggp1, xws1, degT, W2, b1.reshape(1, HID))
    aggp2 = _sc_agg(xws2, src4, dst4)
    h2 = _tc_h2(aggp2, xws2, degT, b2.reshape(1, HID))

    att3 = action_lookup_table[:, 0].reshape(NW, _APW, AWIN)
    dfd3 = action_lookup_table[:, 1].reshape(NW, _APW, AWIN)
    att_e, dfd_e = _sc_take(h2, att3, dfd3)

    ns = action_lookup_table[:, 2:3].astype(jnp.float32)
    logits = _tc_mlp(att_e, dfd_e, ns,
                     mw1[0:HID], mw1[HID:2 * HID], mw1[2 * HID:2 * HID + 1],
                     mb1.reshape(1, HID), mw2.reshape(1, HID),
                     mb2.reshape(1, 1))
    return logits.reshape(A)


# speedup vs baseline: 1.0019x; 1.0019x over previous
import functools

import jax
import jax.numpy as jnp
from jax import lax
from jax.experimental import pallas as pl
from jax.experimental.pallas import tpu as pltpu
from jax.experimental.pallas import tpu_sc as plsc

N = 10000
E = 320000
D = 128
HID = 128
A = 16384

NC = 2
NS = 16
NW = NC * NS
NP = 10240
ROWS_PER_TILE = NP // NS
WIN = 128
WPW = 80
CH = 16
NCH = WPW // CH
NBUF = 2
E_PAD = NW * WPW * WIN
PAD = E_PAD - E

_mesh = plsc.VectorSubcoreMesh(core_axis_name="c", subcore_axis_name="s",
                               num_cores=NC, num_subcores=NS)


def _worker_id():
    return lax.axis_index("s") * NC + lax.axis_index("c")


@functools.partial(
    pl.kernel,
    out_type=jax.ShapeDtypeStruct((NC, 1, NP), jnp.float32),
    mesh=_mesh,
    scratch_types=[
        pltpu.VMEM((WPW, WIN), jnp.int32),
        pltpu.VMEM((WIN,), jnp.float32),
        pltpu.VMEM((640,), jnp.float32),
        pltpu.VMEM_SHARED((NP,), jnp.float32),
        pltpu.SemaphoreType.DMA,
    ],
)
def _sc_deg(dst3, out, idx_d, ones, zbuf, acc, sa):
    c = lax.axis_index("c")
    s = lax.axis_index("s")
    wid = _worker_id()

    def _z(i, _):
        zbuf[pl.ds(i * 16, 16)] = jnp.zeros((16,), jnp.float32)
        return 0
    lax.fori_loop(0, 640 // 16, _z, 0)

    def _o(i, _):
        ones[pl.ds(i * 16, 16)] = jnp.ones((16,), jnp.float32)
        return 0
    lax.fori_loop(0, WIN // 16, _o, 0)

    pltpu.sync_copy(zbuf.at[pl.ds(0, ROWS_PER_TILE)],
                    acc.at[pl.ds(s * ROWS_PER_TILE, ROWS_PER_TILE)])
    plsc.subcore_barrier()

    pltpu.sync_copy(dst3.at[wid], idx_d)

    def _step(k, _):
        pltpu.async_copy(ones, acc.at[idx_d.at[k]], sa, add=True)
        return 0
    lax.fori_loop(0, WPW, _step, 0)

    def _drain(k, _):
        pltpu.make_async_copy(ones, acc.at[idx_d.at[k]], sa).wait()
        return 0
    lax.fori_loop(0, WPW, _drain, 0)

    plsc.subcore_barrier()
    @pl.when(s == 0)
    def _():
        pltpu.sync_copy(acc, out.at[c, 0])


@functools.partial(
    pl.kernel,
    out_type=jax.ShapeDtypeStruct((NC, NP, D), jnp.float32),
    mesh=_mesh,
    scratch_types=[
        pltpu.VMEM((2, CH, WIN), jnp.int32),
        pltpu.VMEM((2, CH, WIN), jnp.int32),
        pltpu.VMEM((NBUF, WIN, D), jnp.float32),
        pltpu.VMEM_SHARED((NP, D), jnp.float32),
        pltpu.SemaphoreType.DMA,
        pltpu.SemaphoreType.DMA,
        pltpu.SemaphoreType.DMA,
    ],
)
def _sc_agg(xws, src4, dst4, out, idxs, idxd, rows, acc, sg0, sg1, si):
    c = lax.axis_index("c")
    s = lax.axis_index("s")
    wid = _worker_id()
    sems = (sg0, sg1)

    pend = (pltpu.async_copy(src4.at[wid, 0], idxs.at[0], si),
            pltpu.async_copy(dst4.at[wid, 0], idxd.at[0], si))

    def _z(i, _):
        def _z2(j, _):
            rows[0, i, pl.ds(j * 16, 16)] = jnp.zeros((16,), jnp.float32)
            return 0
        return lax.fori_loop(0, D // 16, _z2, 0)
    lax.fori_loop(0, WIN, _z, 0)

    base = s * ROWS_PER_TILE

    def _zc(i, _):
        pltpu.sync_copy(rows.at[0], acc.at[pl.ds(base + i * WIN, WIN)])
        return 0
    lax.fori_loop(0, ROWS_PER_TILE // WIN, _zc, 0)
    pend[0].wait()
    pend[1].wait()
    plsc.subcore_barrier()

    for b in range(NBUF):
        pltpu.async_copy(xws.at[idxs.at[0, b]], rows.at[b], sems[b])
    pend = (pltpu.async_copy(src4.at[wid, 1], idxs.at[1], si),
            pltpu.async_copy(dst4.at[wid, 1], idxd.at[1], si))

    for ch in range(NCH):
        p = ch % 2

        def _quad(j, _):
            for b in range(NBUF):
                w = NBUF * j + b
                pltpu.make_async_copy(xws.at[idxs.at[p, w]], rows.at[b],
                                      sems[b]).wait()
                pltpu.sync_copy(rows.at[b], acc.at[idxd.at[p, w]],
                                add=True)
                pltpu.async_copy(xws.at[idxs.at[p, w + NBUF]], rows.at[b],
                                 sems[b])
            return 0
        lax.fori_loop(0, CH // NBUF - 1, _quad, 0)

        for b in range(NBUF):
            w = CH - NBUF + b
            pltpu.make_async_copy(xws.at[idxs.at[p, w]], rows.at[b],
                                  sems[b]).wait()
            pltpu.sync_copy(rows.at[b], acc.at[idxd.at[p, w]], add=True)
            if ch + 1 < NCH:
                if b == 0:
                    pend[0].wait()
                    pend[1].wait()
                pltpu.async_copy(xws.at[idxs.at[1 - p, b]], rows.at[b],
                                 sems[b])
                if b == NBUF - 1 and ch + 2 < NCH:
                    pend = (pltpu.async_copy(src4.at[wid, ch + 2],
                                             idxs.at[p], si),
                            pltpu.async_copy(dst4.at[wid, ch + 2],
                                             idxd.at[p], si))

    plsc.subcore_barrier()
    pltpu.sync_copy(acc.at[pl.ds(base, ROWS_PER_TILE)],
                    out.at[c, pl.ds(base, ROWS_PER_TILE)])


AWIN = 128
_AROWS = A // AWIN
_APW = _AROWS // NW


@functools.partial(
    pl.kernel,
    out_type=(jax.ShapeDtypeStruct((A, D), jnp.float32),
              jax.ShapeDtypeStruct((A, D), jnp.float32)),
    mesh=_mesh,
    scratch_types=[
        pltpu.VMEM((_APW, AWIN), jnp.int32),
        pltpu.VMEM((_APW, AWIN), jnp.int32),
        pltpu.VMEM((2, AWIN, D), jnp.float32),
        pltpu.SemaphoreType.DMA,
        pltpu.SemaphoreType.DMA,
        pltpu.SemaphoreType.DMA,
        pltpu.SemaphoreType.DMA,
    ],
)
def _sc_take(h2, att3, dfd3, out_a, out_d, idx_a, idx_d, rows,
             sg0, sg1, sw0, sw1):
    wid = _worker_id()
    pltpu.sync_copy(att3.at[wid], idx_a)
    pltpu.sync_copy(dfd3.at[wid], idx_d)
    sg = (sg0, sg1)
    sw = (sw0, sw1)

    def _idx(t):
        return (idx_a if t % 2 == 0 else idx_d).at[t // 2]

    def _out(t):
        o = out_a if t % 2 == 0 else out_d
        return o.at[pl.ds((wid * _APW + t // 2) * AWIN, AWIN)]

    njobs = 2 * _APW
    pltpu.async_copy(h2.at[_idx(0)], rows.at[0], sg[0])
    for t in range(njobs):
        b = t % 2
        pltpu.make_async_copy(h2.at[_idx(t)], rows.at[b], sg[b]).wait()
        if t + 1 < njobs:
            if t >= 1:
                pltpu.make_async_copy(rows.at[1 - b], _out(t - 1),
                                      sw[1 - b]).wait()
            pltpu.async_copy(h2.at[_idx(t + 1)], rows.at[1 - b], sg[1 - b])
        pltpu.async_copy(rows.at[b], _out(t), sw[b])
    pltpu.make_async_copy(rows.at[0], _out(njobs - 2), sw[0]).wait()
    pltpu.make_async_copy(rows.at[1], _out(njobs - 1), sw[1]).wait()


_RB = 2000


def _dinv_block(dr):
    return lax.rsqrt(dr[:, 0:1] + dr[:, 1:2] + 1.0)


def _tc_xws1_body(xr, wr, dr, outr):
    dinv = _dinv_block(dr)
    outr[...] = jnp.dot(xr[...], wr[...],
                        preferred_element_type=jnp.float32) * dinv


def _tc_xws1(x, W1, degT):
    return pl.pallas_call(
        _tc_xws1_body,
        grid=(N // _RB,),
        in_specs=[
            pl.BlockSpec((_RB, D), lambda i: (i, 0)),
            pl.BlockSpec((D, HID), lambda i: (0, 0)),
            pl.BlockSpec((_RB, NC), lambda i: (i, 0)),
        ],
        out_specs=pl.BlockSpec((_RB, HID), lambda i: (i, 0)),
        out_shape=jax.ShapeDtypeStruct((N, HID), jnp.float32),
    )(x, W1, degT)


def _tc_layer_body(ar, xr, dr, wr, br, outr):
    dinv = _dinv_block(dr)
    h = jnp.maximum(dinv * (ar[0] + ar[1] + xr[...]) + br[...], 0.0)
    outr[...] = jnp.dot(h, wr[...], preferred_element_type=jnp.float32) * dinv


def _tc_layer(aggp, xws, degT, W, b):
    return pl.pallas_call(
        _tc_layer_body,
        grid=(N // _RB,),
        in_specs=[
            pl.BlockSpec((NC, _RB, HID), lambda i: (0, i, 0)),
            pl.BlockSpec((_RB, HID), lambda i: (i, 0)),
            pl.BlockSpec((_RB, NC), lambda i: (i, 0)),
            pl.BlockSpec((HID, HID), lambda i: (0, 0)),
            pl.BlockSpec((1, HID), lambda i: (0, 0)),
        ],
        out_specs=pl.BlockSpec((_RB, HID), lambda i: (i, 0)),
        out_shape=jax.ShapeDtypeStruct((N, HID), jnp.float32),
    )(aggp, xws, degT, W, b)


def _tc_h2_body(ar, xr, dr, br, outr):
    dinv = _dinv_block(dr)
    outr[...] = jnp.maximum(dinv * (ar[0] + ar[1] + xr[...]) + br[...], 0.0)


def _tc_h2(aggp, xws, degT, b):
    return pl.pallas_call(
        _tc_h2_body,
        grid=(N // _RB,),
        in_specs=[
            pl.BlockSpec((NC, _RB, HID), lambda i: (0, i, 0)),
            pl.BlockSpec((_RB, HID), lambda i: (i, 0)),
            pl.BlockSpec((_RB, NC), lambda i: (i, 0)),
            pl.BlockSpec((1, HID), lambda i: (0, 0)),
        ],
        out_specs=pl.BlockSpec((_RB, HID), lambda i: (i, 0)),
        out_shape=jax.ShapeDtypeStruct((N, HID), jnp.float32),
    )(aggp, xws, degT, b)


_AB = 2048


def _bf16_round(v):
    return v.astype(jnp.bfloat16).astype(jnp.float32)


def _tc_mlp_body(atr, dfr, nsr, wa, wb, wc, b1r, w2r, b2r, outr):
    hid = (jnp.dot(atr[...], wa[...], preferred_element_type=jnp.float32)
           + jnp.dot(dfr[...], wb[...], preferred_element_type=jnp.float32)
           + _bf16_round(nsr[...]) * _bf16_round(wc[...]) + b1r[...])
    hid = jnp.maximum(hid, 0.0)
    prod = _bf16_round(hid) * _bf16_round(w2r[...])
    outr[...] = jnp.sum(prod, axis=1, keepdims=True) + b2r[...]


def _tc_mlp(att_e, dfd_e, ns, mw1a, mw1b, mw1c, mb1, mw2r, mb2):
    return pl.pallas_call(
        _tc_mlp_body,
        grid=(A // _AB,),
        in_specs=[
            pl.BlockSpec((_AB, HID), lambda i: (i, 0)),
            pl.BlockSpec((_AB, HID), lambda i: (i, 0)),
            pl.BlockSpec((_AB, 1), lambda i: (i, 0)),
            pl.BlockSpec((HID, HID), lambda i: (0, 0)),
            pl.BlockSpec((HID, HID), lambda i: (0, 0)),
            pl.BlockSpec((1, HID), lambda i: (0, 0)),
            pl.BlockSpec((1, HID), lambda i: (0, 0)),
            pl.BlockSpec((1, HID), lambda i: (0, 0)),
            pl.BlockSpec((1, 1), lambda i: (0, 0)),
        ],
        out_specs=pl.BlockSpec((_AB, 1), lambda i: (i, 0)),
        out_shape=jax.ShapeDtypeStruct((A, 1), jnp.float32),
    )(att_e, dfd_e, ns, mw1a, mw1b, mw1c, mb1, mw2r, mb2)


def kernel(x, edge_index, action_lookup_table, W1, b1, W2, b2,
           skip_attack, skip_defend, mw1, mb1, mw2, mb2):
    pad_src = jnp.arange(PAD, dtype=jnp.int32) % N
    pad_dst = N + jnp.arange(PAD, dtype=jnp.int32) % (NP - N)
    src3 = jnp.concatenate([edge_index[0], pad_src]).reshape(NW, WPW, WIN)
    dst3 = jnp.concatenate([edge_index[1], pad_dst]).reshape(NW, WPW, WIN)
    src4 = src3.reshape(NW, NCH, CH, WIN)
    dst4 = dst3.reshape(NW, NCH, CH, WIN)

    degp = _sc_deg(dst3)
    degT = degp.reshape(NC, NP).T

    xws1 = _tc_xws1(x, W1, degT)
    aggp1 = _sc_agg(xws1, src4, dst4)
    xws2 = _tc_layer(aggp1, xws1, degT, W2, b1.reshape(1, HID))
    aggp2 = _sc_agg(xws2, src4, dst4)
    h2 = _tc_h2(aggp2, xws2, degT, b2.reshape(1, HID))

    att3 = action_lookup_table[:, 0].reshape(NW, _APW, AWIN)
    dfd3 = action_lookup_table[:, 1].reshape(NW, _APW, AWIN)
    att_e, dfd_e = _sc_take(h2, att3, dfd3)

    ns = action_lookup_table[:, 2:3].astype(jnp.float32)
    logits = _tc_mlp(att_e, dfd_e, ns,
                     mw1[0:HID], mw1[HID:2 * HID], mw1[2 * HID:2 * HID + 1],
                     mb1.reshape(1, HID), mw2.reshape(1, HID),
                     mb2.reshape(1, 1))
    return logits.reshape(A)
